# baseline (speedup 1.0000x reference)
import functools

import numpy as np
import jax
import jax.numpy as jnp
from jax import lax
from jax.experimental import pallas as pl
from jax.experimental.pallas import tpu as pltpu
from jax.experimental.pallas import tpu_sc as plsc

D = 128
DW = D // 2
K = 32
OUT = 384
NC = 2
NS = 16
NW = NC * NS
BP = 10240
C = 8
CK = C * K
N_TBL = 10000
CB0 = 640
CB1 = 0
CBMAX = max(CB0, CB1)

_p = np.arange(OUT // 3)
_j, _o = _p // 32, _p % 32
PERM = np.where(_o < 16, 32 * _j + 2 * _o, 32 * _j + 2 * (_o - 16) + 1)


def _sc_gather_mean(nodes_p, adj2d, dis2d, table_i32):
    mesh = plsc.VectorSubcoreMesh(core_axis_name="c", subcore_axis_name="s",
                                  num_cores=NC, num_subcores=NS)

    @functools.partial(
        pl.kernel,
        out_type=(
            jax.ShapeDtypeStruct((BP, DW), jnp.int32),
            jax.ShapeDtypeStruct((BP, D), jnp.float32),
            jax.ShapeDtypeStruct((BP, D), jnp.float32),
        ),
        mesh=mesh,
        compiler_params=pltpu.CompilerParams(use_tc_tiling_on_sc=False),
        scratch_types=[
            pltpu.VMEM((CBMAX // C, C), jnp.int32),
            pltpu.VMEM((CBMAX // 4, 128), jnp.int32),
            pltpu.VMEM((CBMAX // 4, 128), jnp.int32),
            [pltpu.VMEM((C, DW), jnp.int32)] * 2,
            [pltpu.VMEM((CK, DW), jnp.int32)] * 2,
            [pltpu.VMEM((CK, DW), jnp.int32)] * 2,
            [pltpu.VMEM((C, D), jnp.float32)] * 2,
            [pltpu.VMEM((C, D), jnp.float32)] * 2,
            [pltpu.SemaphoreType.DMA] * 2,
            [pltpu.SemaphoreType.DMA] * 2,
        ],
    )
    def sc_kernel(nodes_hbm, adj_hbm, dis_hbm, tbl_hbm,
                  self_out, adj_out, dis_out,
                  nidx, aidx, didx, srows, arows, drows, amean, dmean,
                  gsem, wsem):
        sid = lax.axis_index("s")

        def run_worker(nbase, nchunk):
            with jax.named_scope("stage_idx"):
                pltpu.sync_copy(nodes_hbm.at[pl.ds(nbase // C, nchunk)],
                                nidx.at[pl.ds(0, nchunk)])
                pltpu.sync_copy(adj_hbm.at[pl.ds(nbase // 4, 2 * nchunk)],
                                aidx.at[pl.ds(0, 2 * nchunk)])
                pltpu.sync_copy(dis_hbm.at[pl.ds(nbase // 4, 2 * nchunk)],
                                didx.at[pl.ds(0, 2 * nchunk)])

            def gather_cps(c, s, make_only):
                mk = pltpu.make_async_copy if make_only else pltpu.async_copy
                return [
                    mk(tbl_hbm.at[nidx.at[c]], srows[s], gsem[s]),
                    mk(tbl_hbm.at[aidx.at[2 * c]],
                       arows[s].at[pl.ds(0, 128)], gsem[s]),
                    mk(tbl_hbm.at[aidx.at[2 * c + 1]],
                       arows[s].at[pl.ds(128, 128)], gsem[s]),
                    mk(tbl_hbm.at[didx.at[2 * c]],
                       drows[s].at[pl.ds(0, 128)], gsem[s]),
                    mk(tbl_hbm.at[didx.at[2 * c + 1]],
                       drows[s].at[pl.ds(128, 128)], gsem[s]),
                ]

            def wb_cps(base, s, make_only):
                mk = pltpu.make_async_copy if make_only else pltpu.async_copy
                return [
                    mk(srows[s], self_out.at[pl.ds(base, C)], wsem[s]),
                    mk(amean[s], adj_out.at[pl.ds(base, C)], wsem[s]),
                    mk(dmean[s], dis_out.at[pl.ds(base, C)], wsem[s]),
                ]

            gather_cps(0, 0, False)

            inv_k = jnp.full((16,), 1.0 / K, jnp.float32)

            def pair_body(t, carry):
                for s in range(2):
                    c = 2 * t + s
                    base = nbase + c * C

                    with jax.named_scope("fire"):
                        @pl.when(c + 1 < nchunk)
                        def _():
                            gather_cps(c + 1, 1 - s, False)

                    with jax.named_scope("gwait"):
                        for cp in gather_cps(c, s, True):
                            cp.wait()

                    with jax.named_scope("wbwait"):
                        @pl.when(c >= 2)
                        def _():
                            for cp in wb_cps(base, s, True):
                                cp.wait()

                    ar, dr = arows[s], drows[s]
                    am, dm = amean[s], dmean[s]
                    himask = jnp.full((16,), -65536, jnp.int32)

                    def node_body(i, carry2):
                        rb = i * K

                        def dbody(d, carry3):
                            sl = pl.ds(d * 16, 16)

                            def kacc(k2, accs):
                                aa, ab, da, db = accs
                                for u in range(2):
                                    r = rb + k2 * 2 + u
                                    wa = ar[r, sl]
                                    wd = dr[r, sl]
                                    aa = aa + lax.bitcast_convert_type(
                                        wa << 16, jnp.float32)
                                    ab = ab + lax.bitcast_convert_type(
                                        wa & himask, jnp.float32)
                                    da = da + lax.bitcast_convert_type(
                                        wd << 16, jnp.float32)
                                    db = db + lax.bitcast_convert_type(
                                        wd & himask, jnp.float32)
                                return (aa, ab, da, db)

                            zero = jnp.zeros((16,), jnp.float32)
                            aa, ab, da, db = lax.fori_loop(
                                0, K // 2, kacc, (zero, zero, zero, zero))
                            am[i, pl.ds(d * 32, 16)] = aa * inv_k
                            am[i, pl.ds(d * 32 + 16, 16)] = ab * inv_k
                            dm[i, pl.ds(d * 32, 16)] = da * inv_k
                            dm[i, pl.ds(d * 32 + 16, 16)] = db * inv_k
                            return carry3

                        lax.fori_loop(0, 4, dbody, 0)
                        return carry2

                    with jax.named_scope("acc"):
                        lax.fori_loop(0, C, node_body, 0)
                    wb_cps(base, s, False)
                return carry

            lax.fori_loop(0, nchunk // 2, pair_body, 0)

            for s in range(2):
                base = nbase + (nchunk - 2 + s) * C
                for cp in wb_cps(base, s, True):
                    cp.wait()

        @pl.when(lax.axis_index("c") == 0)
        def _():
            run_worker(sid * CB0, CB0 // C)

        if CB1:
            @pl.when(lax.axis_index("c") == 1)
            def _():
                run_worker(NS * CB0 + sid * CB1, CB1 // C)

    return sc_kernel(nodes_p, adj2d, dis2d, table_i32)


def _tc_finish(selfs_bf, adjm, dism, wt_self, wt_adj_p, wt_dis_p, bias2d):
    BM = 512

    def body(s_ref, a_ref, d_ref, ws_ref, wa_ref, wd_ref, b_ref, o_ref):
        s = s_ref[...].astype(jnp.float32)
        hs = jnp.dot(s, ws_ref[...], preferred_element_type=jnp.float32)
        ha = jnp.dot(a_ref[...], wa_ref[...], preferred_element_type=jnp.float32)
        hd = jnp.dot(d_ref[...], wd_ref[...], preferred_element_type=jnp.float32)
        h = jnp.concatenate([hs, ha, hd], axis=-1) + b_ref[...]
        h = jnp.where(h >= 0, h, 0.2 * h)
        n = jnp.sqrt(jnp.sum(h * h, axis=-1, keepdims=True))
        o_ref[...] = h / jnp.maximum(n, 1e-12)

    return pl.pallas_call(
        body,
        grid=(BP // BM,),
        in_specs=[
            pl.BlockSpec((BM, D), lambda i: (i, 0)),
            pl.BlockSpec((BM, D), lambda i: (i, 0)),
            pl.BlockSpec((BM, D), lambda i: (i, 0)),
            pl.BlockSpec((D, D), lambda i: (0, 0)),
            pl.BlockSpec((D, D), lambda i: (0, 0)),
            pl.BlockSpec((D, D), lambda i: (0, 0)),
            pl.BlockSpec((1, OUT), lambda i: (0, 0)),
        ],
        out_specs=pl.BlockSpec((BM, OUT), lambda i: (i, 0)),
        out_shape=jax.ShapeDtypeStruct((N_TBL, OUT), jnp.float32),
    )(selfs_bf, adjm, dism, wt_self, wt_adj_p, wt_dis_p, bias2d)


def kernel(nodes, adj_neighbors, dis_neighbors, feat_table,
           W_self, W_adj, W_dis, bias):
    b = nodes.shape[0]
    pad = BP - b
    nodes_p = jnp.concatenate(
        [nodes, jnp.zeros((pad,), jnp.int32)]).reshape(BP // C, C)
    adj_p = jnp.concatenate(
        [adj_neighbors, jnp.zeros((pad, K), jnp.int32)]).reshape(BP * K // 128, 128)
    dis_p = jnp.concatenate(
        [dis_neighbors, jnp.zeros((pad, K), jnp.int32)]).reshape(BP * K // 128, 128)

    tbl_bf = feat_table.astype(jnp.bfloat16)
    tbl_i32 = lax.bitcast_convert_type(tbl_bf.reshape(-1, DW, 2), jnp.int32)

    selfw, adjm, dism = _sc_gather_mean(nodes_p, adj_p, dis_p, tbl_i32)
    selfs_bf = lax.bitcast_convert_type(selfw, jnp.bfloat16).reshape(BP, D)

    out = _tc_finish(selfs_bf, adjm, dism,
                     W_self.T, W_adj.T[PERM], W_dis.T[PERM],
                     bias.reshape(1, OUT))
    return out


# speedup vs baseline: 1.9903x; 1.9903x over previous
import functools

import numpy as np
import jax
import jax.numpy as jnp
from jax import lax
from jax.experimental import pallas as pl
from jax.experimental.pallas import tpu as pltpu
from jax.experimental.pallas import tpu_sc as plsc

D = 128
DW = D // 2
K = 32
OUT = 384
NC = 2
NS = 16
NW = NC * NS
BP = 10240
C = 8
CK = C * K
N_TBL = 10000
CB0 = 640
CB1 = 0
CBMAX = max(CB0, CB1)

_p = np.arange(OUT // 3)
_j, _o = _p // 32, _p % 32
PERM = np.where(_o < 16, 32 * _j + 2 * _o, 32 * _j + 2 * (_o - 16) + 1)


def _sc_gather_mean(nodes_p, adj2d, dis2d, table_i32):
    mesh = plsc.VectorSubcoreMesh(core_axis_name="c", subcore_axis_name="s",
                                  num_cores=NC, num_subcores=NS)

    @functools.partial(
        pl.kernel,
        out_type=(
            jax.ShapeDtypeStruct((BP, DW), jnp.int32),
            jax.ShapeDtypeStruct((BP, D), jnp.float32),
            jax.ShapeDtypeStruct((BP, D), jnp.float32),
        ),
        mesh=mesh,
        compiler_params=pltpu.CompilerParams(use_tc_tiling_on_sc=False),
        scratch_types=[
            pltpu.VMEM((CBMAX // C, C), jnp.int32),
            pltpu.VMEM((CBMAX // 4, 128), jnp.int32),
            pltpu.VMEM((CBMAX // 4, 128), jnp.int32),
            [pltpu.VMEM((C, DW), jnp.int32)] * 2,
            [pltpu.VMEM((CK, DW), jnp.int32)] * 2,
            [pltpu.VMEM((CK, DW), jnp.int32)] * 2,
            [pltpu.VMEM((C, D), jnp.float32)] * 2,
            [pltpu.VMEM((C, D), jnp.float32)] * 2,
            [pltpu.SemaphoreType.DMA] * 2,
            [pltpu.SemaphoreType.DMA] * 2,
        ],
    )
    def sc_kernel(nodes_hbm, adj_hbm, dis_hbm, tbl_hbm,
                  self_out, adj_out, dis_out,
                  nidx, aidx, didx, srows, arows, drows, amean, dmean,
                  gsem, wsem):
        sid = lax.axis_index("s")

        def run_worker(nbase, nchunk):
            with jax.named_scope("stage_idx"):
                pltpu.sync_copy(nodes_hbm.at[pl.ds(nbase // C, nchunk)],
                                nidx.at[pl.ds(0, nchunk)])
                pltpu.sync_copy(adj_hbm.at[pl.ds(nbase // 4, 2 * nchunk)],
                                aidx.at[pl.ds(0, 2 * nchunk)])
                pltpu.sync_copy(dis_hbm.at[pl.ds(nbase // 4, 2 * nchunk)],
                                didx.at[pl.ds(0, 2 * nchunk)])

            def gather_cps(c, s, make_only):
                mk = pltpu.make_async_copy if make_only else pltpu.async_copy
                return [
                    mk(tbl_hbm.at[nidx.at[c]], srows[s], gsem[s]),
                    mk(tbl_hbm.at[aidx.at[2 * c]],
                       arows[s].at[pl.ds(0, 128)], gsem[s]),
                    mk(tbl_hbm.at[aidx.at[2 * c + 1]],
                       arows[s].at[pl.ds(128, 128)], gsem[s]),
                    mk(tbl_hbm.at[didx.at[2 * c]],
                       drows[s].at[pl.ds(0, 128)], gsem[s]),
                    mk(tbl_hbm.at[didx.at[2 * c + 1]],
                       drows[s].at[pl.ds(128, 128)], gsem[s]),
                ]

            def wb_cps(base, s, make_only):
                mk = pltpu.make_async_copy if make_only else pltpu.async_copy
                return [
                    mk(srows[s], self_out.at[pl.ds(base, C)], wsem[s]),
                    mk(amean[s], adj_out.at[pl.ds(base, C)], wsem[s]),
                    mk(dmean[s], dis_out.at[pl.ds(base, C)], wsem[s]),
                ]

            gather_cps(0, 0, False)

            inv_k = jnp.full((16,), 1.0 / K, jnp.float32)

            def pair_body(t, carry):
                for s in range(2):
                    c = 2 * t + s
                    base = nbase + c * C

                    with jax.named_scope("fire"):
                        @pl.when(c + 1 < nchunk)
                        def _():
                            gather_cps(c + 1, 1 - s, False)

                    with jax.named_scope("gwait"):
                        for cp in gather_cps(c, s, True):
                            cp.wait()

                    with jax.named_scope("wbwait"):
                        @pl.when(c >= 2)
                        def _():
                            for cp in wb_cps(base, s, True):
                                cp.wait()

                    ar, dr = arows[s], drows[s]
                    am, dm = amean[s], dmean[s]
                    himask = jnp.full((16,), -65536, jnp.int32)

                    def node_body(i, carry2):
                        rb = i * K

                        def dbody(d, carry3):
                            sl = pl.ds(d * 16, 16)

                            def kacc(k2, accs):
                                aa, ab, da, db = accs
                                for u in range(2):
                                    r = rb + k2 * 2 + u
                                    wa = ar[r, sl]
                                    wd = dr[r, sl]
                                    aa = aa + lax.bitcast_convert_type(
                                        wa << 16, jnp.float32)
                                    ab = ab + lax.bitcast_convert_type(
                                        wa & himask, jnp.float32)
                                    da = da + lax.bitcast_convert_type(
                                        wd << 16, jnp.float32)
                                    db = db + lax.bitcast_convert_type(
                                        wd & himask, jnp.float32)
                                return (aa, ab, da, db)

                            zero = jnp.zeros((16,), jnp.float32)
                            aa, ab, da, db = lax.fori_loop(
                                0, K // 2, kacc, (zero, zero, zero, zero))
                            am[i, pl.ds(d * 32, 16)] = aa * inv_k
                            am[i, pl.ds(d * 32 + 16, 16)] = ab * inv_k
                            dm[i, pl.ds(d * 32, 16)] = da * inv_k
                            dm[i, pl.ds(d * 32 + 16, 16)] = db * inv_k
                            return carry3

                        lax.fori_loop(0, 4, dbody, 0)
                        return carry2

                    with jax.named_scope("acc"):
                        lax.fori_loop(0, C, node_body, 0)
                    wb_cps(base, s, False)
                return carry

            lax.fori_loop(0, nchunk // 2, pair_body, 0)

            for s in range(2):
                base = nbase + (nchunk - 2 + s) * C
                for cp in wb_cps(base, s, True):
                    cp.wait()

        @pl.when(lax.axis_index("c") == 0)
        def _():
            run_worker(sid * CB0, CB0 // C)

        if CB1:
            @pl.when(lax.axis_index("c") == 1)
            def _():
                run_worker(NS * CB0 + sid * CB1, CB1 // C)

    return sc_kernel(nodes_p, adj2d, dis2d, table_i32)


def _tc_finish(selfs_bf, adjm, dism, wt_self, wt_adj_p, wt_dis_p, bias2d):
    BM = 512

    def body(s_ref, a_ref, d_ref, ws_ref, wa_ref, wd_ref, b_ref, o_ref):
        s = s_ref[...].astype(jnp.float32)
        hs = jnp.dot(s, ws_ref[...], preferred_element_type=jnp.float32)
        ha = jnp.dot(a_ref[...], wa_ref[...], preferred_element_type=jnp.float32)
        hd = jnp.dot(d_ref[...], wd_ref[...], preferred_element_type=jnp.float32)
        h = jnp.concatenate([hs, ha, hd], axis=-1) + b_ref[...]
        h = jnp.where(h >= 0, h, 0.2 * h)
        n = jnp.sqrt(jnp.sum(h * h, axis=-1, keepdims=True))
        o_ref[...] = h / jnp.maximum(n, 1e-12)

    return pl.pallas_call(
        body,
        grid=(BP // BM,),
        in_specs=[
            pl.BlockSpec((BM, D), lambda i: (i, 0)),
            pl.BlockSpec((BM, D), lambda i: (i, 0)),
            pl.BlockSpec((BM, D), lambda i: (i, 0)),
            pl.BlockSpec((D, D), lambda i: (0, 0)),
            pl.BlockSpec((D, D), lambda i: (0, 0)),
            pl.BlockSpec((D, D), lambda i: (0, 0)),
            pl.BlockSpec((1, OUT), lambda i: (0, 0)),
        ],
        out_specs=pl.BlockSpec((BM, OUT), lambda i: (i, 0)),
        out_shape=jax.ShapeDtypeStruct((N_TBL, OUT), jnp.float32),
    )(selfs_bf, adjm, dism, wt_self, wt_adj_p, wt_dis_p, bias2d)


def kernel(nodes, adj_neighbors, dis_neighbors, feat_table,
           W_self, W_adj, W_dis, bias):
    b = nodes.shape[0]
    pad = BP - b
    pad_n = (jnp.arange(pad, dtype=jnp.int32) * 41) % N_TBL
    pad_k = (jnp.arange(pad * K, dtype=jnp.int32) * 41).reshape(pad, K) % N_TBL
    nodes_p = jnp.concatenate([nodes, pad_n]).reshape(BP // C, C)
    adj_p = jnp.concatenate(
        [adj_neighbors, pad_k]).reshape(BP * K // 128, 128)
    dis_p = jnp.concatenate(
        [dis_neighbors, pad_k]).reshape(BP * K // 128, 128)

    tbl_bf = feat_table.astype(jnp.bfloat16)
    tbl_i32 = lax.bitcast_convert_type(tbl_bf.reshape(-1, DW, 2), jnp.int32)

    selfw, adjm, dism = _sc_gather_mean(nodes_p, adj_p, dis_p, tbl_i32)
    selfs_bf = lax.bitcast_convert_type(selfw, jnp.bfloat16).reshape(BP, D)

    out = _tc_finish(selfs_bf, adjm, dism,
                     W_self.T, W_adj.T[PERM], W_dis.T[PERM],
                     bias.reshape(1, OUT))
    return out


# speedup vs baseline: 2.7592x; 1.3864x over previous
import functools

import numpy as np
import jax
import jax.numpy as jnp
from jax import lax
from jax.experimental import pallas as pl
from jax.experimental.pallas import tpu as pltpu
from jax.experimental.pallas import tpu_sc as plsc

D = 128
DW = D // 2
K = 32
OUT = 384
NC = 2
NS = 16
NW = NC * NS
BP = 10240
C = 8
CK = C * K
N_TBL = 10000
CB0 = 320
CB1 = 320
CBMAX = max(CB0, CB1)

_p = np.arange(OUT // 3)
_j, _o = _p // 32, _p % 32
PERM = np.where(_o < 16, 32 * _j + 2 * _o, 32 * _j + 2 * (_o - 16) + 1)


def _sc_gather_mean(nodes_p, adj2d, dis2d, table_i32):
    mesh = plsc.VectorSubcoreMesh(core_axis_name="c", subcore_axis_name="s",
                                  num_cores=NC, num_subcores=NS)

    @functools.partial(
        pl.kernel,
        out_type=(
            jax.ShapeDtypeStruct((BP, DW), jnp.int32),
            jax.ShapeDtypeStruct((BP, D), jnp.float32),
            jax.ShapeDtypeStruct((BP, D), jnp.float32),
        ),
        mesh=mesh,
        compiler_params=pltpu.CompilerParams(use_tc_tiling_on_sc=False),
        scratch_types=[
            pltpu.VMEM((CBMAX // C, C), jnp.int32),
            pltpu.VMEM((CBMAX // 4, 128), jnp.int32),
            pltpu.VMEM((CBMAX // 4, 128), jnp.int32),
            [pltpu.VMEM((C, DW), jnp.int32)] * 2,
            [pltpu.VMEM((CK, DW), jnp.int32)] * 2,
            [pltpu.VMEM((CK, DW), jnp.int32)] * 2,
            [pltpu.VMEM((C, D), jnp.float32)] * 2,
            [pltpu.VMEM((C, D), jnp.float32)] * 2,
            [pltpu.SemaphoreType.DMA] * 2,
            [pltpu.SemaphoreType.DMA] * 2,
        ],
    )
    def sc_kernel(nodes_hbm, adj_hbm, dis_hbm, tbl_hbm,
                  self_out, adj_out, dis_out,
                  nidx, aidx, didx, srows, arows, drows, amean, dmean,
                  gsem, wsem):
        sid = lax.axis_index("s")

        def run_worker(nbase, nchunk):
            with jax.named_scope("stage_idx"):
                pltpu.sync_copy(nodes_hbm.at[pl.ds(nbase // C, nchunk)],
                                nidx.at[pl.ds(0, nchunk)])
                pltpu.sync_copy(adj_hbm.at[pl.ds(nbase // 4, 2 * nchunk)],
                                aidx.at[pl.ds(0, 2 * nchunk)])
                pltpu.sync_copy(dis_hbm.at[pl.ds(nbase // 4, 2 * nchunk)],
                                didx.at[pl.ds(0, 2 * nchunk)])

            def gather_cps(c, s, make_only):
                mk = pltpu.make_async_copy if make_only else pltpu.async_copy
                return [
                    mk(tbl_hbm.at[nidx.at[c]], srows[s], gsem[s]),
                    mk(tbl_hbm.at[aidx.at[2 * c]],
                       arows[s].at[pl.ds(0, 128)], gsem[s]),
                    mk(tbl_hbm.at[aidx.at[2 * c + 1]],
                       arows[s].at[pl.ds(128, 128)], gsem[s]),
                    mk(tbl_hbm.at[didx.at[2 * c]],
                       drows[s].at[pl.ds(0, 128)], gsem[s]),
                    mk(tbl_hbm.at[didx.at[2 * c + 1]],
                       drows[s].at[pl.ds(128, 128)], gsem[s]),
                ]

            def wb_cps(base, s, make_only):
                mk = pltpu.make_async_copy if make_only else pltpu.async_copy
                return [
                    mk(srows[s], self_out.at[pl.ds(base, C)], wsem[s]),
                    mk(amean[s], adj_out.at[pl.ds(base, C)], wsem[s]),
                    mk(dmean[s], dis_out.at[pl.ds(base, C)], wsem[s]),
                ]

            gather_cps(0, 0, False)

            inv_k = jnp.full((16,), 1.0 / K, jnp.float32)

            def pair_body(t, carry):
                for s in range(2):
                    c = 2 * t + s
                    base = nbase + c * C

                    with jax.named_scope("fire"):
                        @pl.when(c + 1 < nchunk)
                        def _():
                            gather_cps(c + 1, 1 - s, False)

                    with jax.named_scope("gwait"):
                        for cp in gather_cps(c, s, True):
                            cp.wait()

                    with jax.named_scope("wbwait"):
                        @pl.when(c >= 2)
                        def _():
                            for cp in wb_cps(base, s, True):
                                cp.wait()

                    ar, dr = arows[s], drows[s]
                    am, dm = amean[s], dmean[s]
                    himask = jnp.full((16,), -65536, jnp.int32)

                    def node_body(i, carry2):
                        rb = i * K

                        def dbody(d, carry3):
                            sl = pl.ds(d * 16, 16)

                            def kacc(k2, accs):
                                aa, ab, da, db = accs
                                for u in range(2):
                                    r = rb + k2 * 2 + u
                                    wa = ar[r, sl]
                                    wd = dr[r, sl]
                                    aa = aa + lax.bitcast_convert_type(
                                        wa << 16, jnp.float32)
                                    ab = ab + lax.bitcast_convert_type(
                                        wa & himask, jnp.float32)
                                    da = da + lax.bitcast_convert_type(
                                        wd << 16, jnp.float32)
                                    db = db + lax.bitcast_convert_type(
                                        wd & himask, jnp.float32)
                                return (aa, ab, da, db)

                            zero = jnp.zeros((16,), jnp.float32)
                            aa, ab, da, db = lax.fori_loop(
                                0, K // 2, kacc, (zero, zero, zero, zero))
                            am[i, pl.ds(d * 32, 16)] = aa * inv_k
                            am[i, pl.ds(d * 32 + 16, 16)] = ab * inv_k
                            dm[i, pl.ds(d * 32, 16)] = da * inv_k
                            dm[i, pl.ds(d * 32 + 16, 16)] = db * inv_k
                            return carry3

                        lax.fori_loop(0, 4, dbody, 0)
                        return carry2

                    with jax.named_scope("acc"):
                        lax.fori_loop(0, C, node_body, 0)
                    wb_cps(base, s, False)
                return carry

            lax.fori_loop(0, nchunk // 2, pair_body, 0)

            for s in range(2):
                base = nbase + (nchunk - 2 + s) * C
                for cp in wb_cps(base, s, True):
                    cp.wait()

        @pl.when(lax.axis_index("c") == 0)
        def _():
            run_worker(sid * CB0, CB0 // C)

        if CB1:
            @pl.when(lax.axis_index("c") == 1)
            def _():
                run_worker(NS * CB0 + sid * CB1, CB1 // C)

    return sc_kernel(nodes_p, adj2d, dis2d, table_i32)


def _tc_finish(selfs_bf, adjm, dism, wt_self, wt_adj_p, wt_dis_p, bias2d):
    BM = 512

    def body(s_ref, a_ref, d_ref, ws_ref, wa_ref, wd_ref, b_ref, o_ref):
        s = s_ref[...].astype(jnp.float32)
        hs = jnp.dot(s, ws_ref[...], preferred_element_type=jnp.float32)
        ha = jnp.dot(a_ref[...], wa_ref[...], preferred_element_type=jnp.float32)
        hd = jnp.dot(d_ref[...], wd_ref[...], preferred_element_type=jnp.float32)
        h = jnp.concatenate([hs, ha, hd], axis=-1) + b_ref[...]
        h = jnp.where(h >= 0, h, 0.2 * h)
        n = jnp.sqrt(jnp.sum(h * h, axis=-1, keepdims=True))
        o_ref[...] = h / jnp.maximum(n, 1e-12)

    return pl.pallas_call(
        body,
        grid=(BP // BM,),
        in_specs=[
            pl.BlockSpec((BM, D), lambda i: (i, 0)),
            pl.BlockSpec((BM, D), lambda i: (i, 0)),
            pl.BlockSpec((BM, D), lambda i: (i, 0)),
            pl.BlockSpec((D, D), lambda i: (0, 0)),
            pl.BlockSpec((D, D), lambda i: (0, 0)),
            pl.BlockSpec((D, D), lambda i: (0, 0)),
            pl.BlockSpec((1, OUT), lambda i: (0, 0)),
        ],
        out_specs=pl.BlockSpec((BM, OUT), lambda i: (i, 0)),
        out_shape=jax.ShapeDtypeStruct((N_TBL, OUT), jnp.float32),
    )(selfs_bf, adjm, dism, wt_self, wt_adj_p, wt_dis_p, bias2d)


def kernel(nodes, adj_neighbors, dis_neighbors, feat_table,
           W_self, W_adj, W_dis, bias):
    b = nodes.shape[0]
    pad = BP - b
    pad_n = (jnp.arange(pad, dtype=jnp.int32) * 41) % N_TBL
    pad_k = (jnp.arange(pad * K, dtype=jnp.int32) * 41).reshape(pad, K) % N_TBL
    nodes_p = jnp.concatenate([nodes, pad_n]).reshape(BP // C, C)
    adj_p = jnp.concatenate(
        [adj_neighbors, pad_k]).reshape(BP * K // 128, 128)
    dis_p = jnp.concatenate(
        [dis_neighbors, pad_k]).reshape(BP * K // 128, 128)

    tbl_bf = feat_table.astype(jnp.bfloat16)
    tbl_i32 = lax.bitcast_convert_type(tbl_bf.reshape(-1, DW, 2), jnp.int32)

    selfw, adjm, dism = _sc_gather_mean(nodes_p, adj_p, dis_p, tbl_i32)
    selfs_bf = lax.bitcast_convert_type(selfw, jnp.bfloat16).reshape(BP, D)

    out = _tc_finish(selfs_bf, adjm, dism,
                     W_self.T, W_adj.T[PERM], W_dis.T[PERM],
                     bias.reshape(1, OUT))
    return out


# speedup vs baseline: 2.8175x; 1.0211x over previous
import functools

import numpy as np
import jax
import jax.numpy as jnp
from jax import lax
from jax.experimental import pallas as pl
from jax.experimental.pallas import tpu as pltpu
from jax.experimental.pallas import tpu_sc as plsc

D = 128
DW = D // 2
K = 32
OUT = 384
NC = 2
NS = 16
NW = NC * NS
BP = 10240
C = 8
CK = C * K
N_TBL = 10000
CB0 = 320
CB1 = 320
CBMAX = max(CB0, CB1)

_p = np.arange(OUT // 3)
_j, _o = _p // 32, _p % 32
PERM = np.where(_o < 16, 32 * _j + 2 * _o, 32 * _j + 2 * (_o - 16) + 1)


def _sc_gather_mean(nodes_p, adj2d, dis2d, table_i32):
    mesh = plsc.VectorSubcoreMesh(core_axis_name="c", subcore_axis_name="s",
                                  num_cores=NC, num_subcores=NS)

    @functools.partial(
        pl.kernel,
        out_type=(
            jax.ShapeDtypeStruct((BP, DW), jnp.int32),
            jax.ShapeDtypeStruct((BP, D), jnp.float32),
            jax.ShapeDtypeStruct((BP, D), jnp.float32),
        ),
        mesh=mesh,
        compiler_params=pltpu.CompilerParams(use_tc_tiling_on_sc=False),
        scratch_types=[
            pltpu.VMEM((CBMAX // C, C), jnp.int32),
            pltpu.VMEM((CBMAX // 4, 128), jnp.int32),
            pltpu.VMEM((CBMAX // 4, 128), jnp.int32),
            [pltpu.VMEM((C, DW), jnp.int32)] * 2,
            [pltpu.VMEM((CK, DW), jnp.int32)] * 2,
            [pltpu.VMEM((CK, DW), jnp.int32)] * 2,
            [pltpu.VMEM((C, D), jnp.float32)] * 2,
            [pltpu.VMEM((C, D), jnp.float32)] * 2,
            [pltpu.SemaphoreType.DMA] * 2,
            [pltpu.SemaphoreType.DMA] * 2,
        ],
    )
    def sc_kernel(nodes_hbm, adj_hbm, dis_hbm, tbl_hbm,
                  self_out, adj_out, dis_out,
                  nidx, aidx, didx, srows, arows, drows, amean, dmean,
                  gsem, wsem):
        sid = lax.axis_index("s")

        def run_worker(nbase, nchunk):
            with jax.named_scope("stage_idx"):
                pltpu.sync_copy(nodes_hbm.at[pl.ds(nbase // C, nchunk)],
                                nidx.at[pl.ds(0, nchunk)])
                pltpu.sync_copy(adj_hbm.at[pl.ds(nbase // 4, 2 * nchunk)],
                                aidx.at[pl.ds(0, 2 * nchunk)])
                pltpu.sync_copy(dis_hbm.at[pl.ds(nbase // 4, 2 * nchunk)],
                                didx.at[pl.ds(0, 2 * nchunk)])

            def gather_cps(c, s, make_only):
                mk = pltpu.make_async_copy if make_only else pltpu.async_copy
                return [
                    mk(tbl_hbm.at[nidx.at[c]], srows[s], gsem[s]),
                    mk(tbl_hbm.at[aidx.at[2 * c]],
                       arows[s].at[pl.ds(0, 128)], gsem[s]),
                    mk(tbl_hbm.at[aidx.at[2 * c + 1]],
                       arows[s].at[pl.ds(128, 128)], gsem[s]),
                    mk(tbl_hbm.at[didx.at[2 * c]],
                       drows[s].at[pl.ds(0, 128)], gsem[s]),
                    mk(tbl_hbm.at[didx.at[2 * c + 1]],
                       drows[s].at[pl.ds(128, 128)], gsem[s]),
                ]

            def wb_cps(base, s, make_only):
                mk = pltpu.make_async_copy if make_only else pltpu.async_copy
                return [
                    mk(srows[s], self_out.at[pl.ds(base, C)], wsem[s]),
                    mk(amean[s], adj_out.at[pl.ds(base, C)], wsem[s]),
                    mk(dmean[s], dis_out.at[pl.ds(base, C)], wsem[s]),
                ]

            gather_cps(0, 0, False)

            inv_k = jnp.full((16,), 1.0 / K, jnp.float32)

            def pair_body(t, carry):
                for s in range(2):
                    c = 2 * t + s
                    base = nbase + c * C

                    with jax.named_scope("fire"):
                        @pl.when(c + 1 < nchunk)
                        def _():
                            gather_cps(c + 1, 1 - s, False)

                    with jax.named_scope("gwait"):
                        for cp in gather_cps(c, s, True):
                            cp.wait()

                    with jax.named_scope("wbwait"):
                        @pl.when(c >= 2)
                        def _():
                            for cp in wb_cps(base, s, True):
                                cp.wait()

                    ar, dr = arows[s], drows[s]
                    am, dm = amean[s], dmean[s]
                    himask = jnp.full((16,), -65536, jnp.int32)

                    def node_body(i, carry2):
                        rb = i * K
                        for d in range(4):
                            sl = pl.ds(d * 16, 16)

                            def kacc(k4, accs):
                                aa, ab, da, db = accs
                                for u in range(4):
                                    r = rb + k4 * 4 + u
                                    wa = ar[r, sl]
                                    wd = dr[r, sl]
                                    aa = aa + lax.bitcast_convert_type(
                                        wa << 16, jnp.float32)
                                    ab = ab + lax.bitcast_convert_type(
                                        wa & himask, jnp.float32)
                                    da = da + lax.bitcast_convert_type(
                                        wd << 16, jnp.float32)
                                    db = db + lax.bitcast_convert_type(
                                        wd & himask, jnp.float32)
                                return (aa, ab, da, db)

                            zero = jnp.zeros((16,), jnp.float32)
                            aa, ab, da, db = lax.fori_loop(
                                0, K // 4, kacc, (zero, zero, zero, zero))
                            am[i, pl.ds(d * 32, 16)] = aa * inv_k
                            am[i, pl.ds(d * 32 + 16, 16)] = ab * inv_k
                            dm[i, pl.ds(d * 32, 16)] = da * inv_k
                            dm[i, pl.ds(d * 32 + 16, 16)] = db * inv_k
                        return carry2

                    with jax.named_scope("acc"):
                        lax.fori_loop(0, C, node_body, 0)
                    wb_cps(base, s, False)
                return carry

            lax.fori_loop(0, nchunk // 2, pair_body, 0)

            for s in range(2):
                base = nbase + (nchunk - 2 + s) * C
                for cp in wb_cps(base, s, True):
                    cp.wait()

        @pl.when(lax.axis_index("c") == 0)
        def _():
            run_worker(sid * CB0, CB0 // C)

        if CB1:
            @pl.when(lax.axis_index("c") == 1)
            def _():
                run_worker(NS * CB0 + sid * CB1, CB1 // C)

    return sc_kernel(nodes_p, adj2d, dis2d, table_i32)


def _tc_finish(selfs_bf, adjm, dism, wt_self, wt_adj_p, wt_dis_p, bias2d):
    BM = 512

    def body(s_ref, a_ref, d_ref, ws_ref, wa_ref, wd_ref, b_ref, o_ref):
        s = s_ref[...].astype(jnp.float32)
        hs = jnp.dot(s, ws_ref[...], preferred_element_type=jnp.float32)
        ha = jnp.dot(a_ref[...], wa_ref[...], preferred_element_type=jnp.float32)
        hd = jnp.dot(d_ref[...], wd_ref[...], preferred_element_type=jnp.float32)
        h = jnp.concatenate([hs, ha, hd], axis=-1) + b_ref[...]
        h = jnp.where(h >= 0, h, 0.2 * h)
        n = jnp.sqrt(jnp.sum(h * h, axis=-1, keepdims=True))
        o_ref[...] = h / jnp.maximum(n, 1e-12)

    return pl.pallas_call(
        body,
        grid=(BP // BM,),
        in_specs=[
            pl.BlockSpec((BM, D), lambda i: (i, 0)),
            pl.BlockSpec((BM, D), lambda i: (i, 0)),
            pl.BlockSpec((BM, D), lambda i: (i, 0)),
            pl.BlockSpec((D, D), lambda i: (0, 0)),
            pl.BlockSpec((D, D), lambda i: (0, 0)),
            pl.BlockSpec((D, D), lambda i: (0, 0)),
            pl.BlockSpec((1, OUT), lambda i: (0, 0)),
        ],
        out_specs=pl.BlockSpec((BM, OUT), lambda i: (i, 0)),
        out_shape=jax.ShapeDtypeStruct((N_TBL, OUT), jnp.float32),
    )(selfs_bf, adjm, dism, wt_self, wt_adj_p, wt_dis_p, bias2d)


def kernel(nodes, adj_neighbors, dis_neighbors, feat_table,
           W_self, W_adj, W_dis, bias):
    b = nodes.shape[0]
    pad = BP - b
    pad_n = (jnp.arange(pad, dtype=jnp.int32) * 41) % N_TBL
    pad_k = (jnp.arange(pad * K, dtype=jnp.int32) * 41).reshape(pad, K) % N_TBL
    nodes_p = jnp.concatenate([nodes, pad_n]).reshape(BP // C, C)
    adj_p = jnp.concatenate(
        [adj_neighbors, pad_k]).reshape(BP * K // 128, 128)
    dis_p = jnp.concatenate(
        [dis_neighbors, pad_k]).reshape(BP * K // 128, 128)

    tbl_bf = feat_table.astype(jnp.bfloat16)
    tbl_i32 = lax.bitcast_convert_type(tbl_bf.reshape(-1, DW, 2), jnp.int32)

    selfw, adjm, dism = _sc_gather_mean(nodes_p, adj_p, dis_p, tbl_i32)
    selfs_bf = lax.bitcast_convert_type(selfw, jnp.bfloat16).reshape(BP, D)

    out = _tc_finish(selfs_bf, adjm, dism,
                     W_self.T, W_adj.T[PERM], W_dis.T[PERM],
                     bias.reshape(1, OUT))
    return out
